# baseline (speedup 1.0000x reference)
import jax
import jax.numpy as jnp
from jax.experimental import pallas as pl
from jax.experimental.pallas import tpu as pltpu


def kernel(x, shape, row_embed, col_embed):
    b, _, h, w = x.shape
    f = row_embed.shape[1]
    hw = h * w
    nsplit = 4
    rows = hw // nsplit

    def body(shape_ref, col_ref, row_ref, out_ref, pos_ref, sem):
        residual = (shape_ref[2] * shape_ref[3] - hw).astype(jnp.float32)
        col = col_ref[...]
        row = row_ref[...]
        pos_ref[:, :f] = jnp.broadcast_to(col[None], (h, w, f)).reshape(hw, f) + residual
        pos_ref[:, f:] = jnp.broadcast_to(row[:, None], (h, w, f)).reshape(hw, f) + residual
        copies = [
            pltpu.make_async_copy(
                pos_ref.at[pl.ds(s * rows, rows)],
                out_ref.at[i, pl.ds(s * rows, rows), :],
                sem.at[i * nsplit + s],
            )
            for i in range(b)
            for s in range(nsplit)
        ]
        for c in copies:
            c.start()
        for c in copies:
            c.wait()

    grid_spec = pltpu.PrefetchScalarGridSpec(
        num_scalar_prefetch=1,
        grid=(1,),
        in_specs=[
            pl.BlockSpec((w, f), lambda i, s: (0, 0)),
            pl.BlockSpec((h, f), lambda i, s: (0, 0)),
        ],
        out_specs=pl.BlockSpec(memory_space=pl.ANY),
        scratch_shapes=[
            pltpu.VMEM((hw, 2 * f), jnp.float32),
            pltpu.SemaphoreType.DMA((b * nsplit,)),
        ],
    )

    return pl.pallas_call(
        body,
        grid_spec=grid_spec,
        out_shape=jax.ShapeDtypeStruct((b, hw, 2 * f), jnp.float32),
    )(shape, col_embed, row_embed)


# speedup vs baseline: 1.0258x; 1.0258x over previous
import jax
import jax.numpy as jnp
from jax.experimental import pallas as pl
from jax.experimental.pallas import tpu as pltpu


def kernel(x, shape, row_embed, col_embed):
    b, _, h, w = x.shape
    f = row_embed.shape[1]
    hw = h * w
    nsplit = 2
    rows = hw // nsplit

    def body(shape_ref, col_ref, row_ref, out_ref, pos_ref, sem):
        residual = (shape_ref[2] * shape_ref[3] - hw).astype(jnp.float32)
        col = col_ref[...]
        row = row_ref[...]
        hs = h // nsplit
        copies = []
        for s in range(nsplit):
            rs = pl.ds(s * rows, rows)
            cols = jnp.broadcast_to(col[None], (hs, w, f)).reshape(rows, f)
            rows_s = jnp.broadcast_to(
                row[s * hs:(s + 1) * hs, None, :], (hs, w, f)
            ).reshape(rows, f)
            pos_ref[rs, :f] = cols + residual
            pos_ref[rs, f:] = rows_s + residual
            new = [
                pltpu.make_async_copy(
                    pos_ref.at[rs],
                    out_ref.at[i, rs, :],
                    sem.at[i * nsplit + s],
                )
                for i in range(b)
            ]
            for c in new:
                c.start()
            copies.extend(new)
        for c in copies:
            c.wait()

    grid_spec = pltpu.PrefetchScalarGridSpec(
        num_scalar_prefetch=1,
        grid=(1,),
        in_specs=[
            pl.BlockSpec((w, f), lambda i, s: (0, 0)),
            pl.BlockSpec((h, f), lambda i, s: (0, 0)),
        ],
        out_specs=pl.BlockSpec(memory_space=pl.ANY),
        scratch_shapes=[
            pltpu.VMEM((hw, 2 * f), jnp.float32),
            pltpu.SemaphoreType.DMA((b * nsplit,)),
        ],
    )

    return pl.pallas_call(
        body,
        grid_spec=grid_spec,
        out_shape=jax.ShapeDtypeStruct((b, hw, 2 * f), jnp.float32),
    )(shape, col_embed, row_embed)


# speedup vs baseline: 1.0390x; 1.0129x over previous
import jax
import jax.numpy as jnp
from jax.experimental import pallas as pl
from jax.experimental.pallas import tpu as pltpu


def kernel(x, shape, row_embed, col_embed):
    b, _, h, w = x.shape
    f = row_embed.shape[1]
    hw = h * w
    nsplit = 2
    rows = hw // nsplit

    def body(shape_ref, col_ref, row_ref, out_ref, pos_ref, sem):
        residual = (shape_ref[2] * shape_ref[3] - hw).astype(jnp.float32)
        col = col_ref[...]
        row = row_ref[...]
        pos_ref[:, :f] = jnp.broadcast_to(col[None], (h, w, f)).reshape(hw, f) + residual
        pos_ref[:, f:] = jnp.broadcast_to(row[:, None], (h, w, f)).reshape(hw, f) + residual
        copies = [
            pltpu.make_async_copy(
                pos_ref.at[pl.ds(s * rows, rows)],
                out_ref.at[i, pl.ds(s * rows, rows), :],
                sem.at[i * nsplit + s],
            )
            for i in range(b)
            for s in range(nsplit)
        ]
        for c in copies:
            c.start()
        for c in copies:
            c.wait()

    grid_spec = pltpu.PrefetchScalarGridSpec(
        num_scalar_prefetch=1,
        grid=(1,),
        in_specs=[
            pl.BlockSpec((w, f), lambda i, s: (0, 0)),
            pl.BlockSpec((h, f), lambda i, s: (0, 0)),
        ],
        out_specs=pl.BlockSpec(memory_space=pl.ANY),
        scratch_shapes=[
            pltpu.VMEM((hw, 2 * f), jnp.float32),
            pltpu.SemaphoreType.DMA((b * nsplit,)),
        ],
    )

    return pl.pallas_call(
        body,
        grid_spec=grid_spec,
        out_shape=jax.ShapeDtypeStruct((b, hw, 2 * f), jnp.float32),
    )(shape, col_embed, row_embed)
